# baseline (speedup 1.0000x reference)
import functools

import jax
import jax.numpy as jnp
from jax.experimental import pallas as pl
from jax.experimental.pallas import tpu as pltpu


def _round_up(v, m):
    return ((v + m - 1) // m) * m


def _pipeline_kernel(x_hbm, w_hbm, b_ref, g_ref, beta_ref, o_hbm,
                     xbuf, obuf, wf32, in_sem, out_sem, w_sem,
                     *, tm, steps, eps, true_out_dim):
    core = pl.program_id(0)
    base = core * steps * tm

    def dma_in(slot, step):
        pltpu.make_async_copy(
            x_hbm.at[pl.ds(base + step * tm, tm), :],
            xbuf.at[slot], in_sem.at[slot]).start()

    def wait_in(slot):
        pltpu.make_async_copy(
            x_hbm.at[pl.ds(0, tm), :],
            xbuf.at[slot], in_sem.at[slot]).wait()

    def dma_out(slot, step):
        pltpu.make_async_copy(
            obuf.at[slot],
            o_hbm.at[pl.ds(base + step * tm, tm), :], out_sem.at[slot]).start()

    def wait_out(slot):
        pltpu.make_async_copy(
            obuf.at[slot],
            o_hbm.at[pl.ds(0, tm), :], out_sem.at[slot]).wait()

    dma_in(0, 0)
    pltpu.make_async_copy(w_hbm, wf32, w_sem).start()
    dma_in(1, 1)
    pltpu.make_async_copy(w_hbm, wf32, w_sem).wait()

    inv_d = 1.0 / float(true_out_dim)

    def compute(slot):
        y = jnp.dot(xbuf[slot], wf32[...], preferred_element_type=jnp.float32)
        y = y + b_ref[...]
        s1 = jnp.sum(y, axis=-1, keepdims=True)
        s2 = jnp.sum(y * y, axis=-1, keepdims=True)
        mean = s1 * inv_d
        var = jnp.maximum(s2 * inv_d - mean * mean, 0.0)
        inv = jax.lax.rsqrt(var + eps)
        out = (y - mean) * inv * g_ref[...] + beta_ref[...]
        obuf[slot] = jnp.maximum(out, 0.0).astype(obuf.dtype)

    for step in range(steps):
        cur = step % 2
        if step + 2 < steps:
            pass
        wait_in(cur)
        if step >= 2:
            wait_out(cur)
        compute(cur)
        dma_out(cur, step)
        if step + 2 < steps:
            dma_in(cur, step + 2)

    wait_out((steps - 2) % 2)
    wait_out((steps - 1) % 2)


def kernel(x, w, b, gamma, beta, *, eps=1e-5):
    n, in_dim = x.shape
    out_dim = w.shape[1]

    in_pad = _round_up(in_dim, 128)
    out_pad = _round_up(out_dim, 128)
    tm = 1024
    n_pad = _round_up(n, 2 * tm)
    steps = n_pad // (2 * tm)

    xp = x
    if (n_pad, in_pad) != x.shape:
        xp = jnp.zeros((n_pad, in_pad), x.dtype).at[:n, :in_dim].set(x)
    wp = w
    if (in_pad, out_pad) != w.shape:
        wp = jnp.zeros((in_pad, out_pad), w.dtype).at[:in_dim, :out_dim].set(w)
    bp = b.astype(jnp.float32)
    gp = gamma.astype(jnp.float32)
    betap = beta.astype(jnp.float32)
    if out_pad != out_dim:
        bp = jnp.zeros((1, out_pad), jnp.float32).at[:, :out_dim].set(bp)
        gp = jnp.ones((1, out_pad), jnp.float32).at[:, :out_dim].set(gp)
        betap = jnp.zeros((1, out_pad), jnp.float32).at[:, :out_dim].set(betap)

    body = functools.partial(_pipeline_kernel, tm=tm, steps=steps, eps=eps,
                             true_out_dim=out_dim)
    y = pl.pallas_call(
        body,
        out_shape=jax.ShapeDtypeStruct((n_pad, out_pad), x.dtype),
        grid=(2,),
        in_specs=[
            pl.BlockSpec(memory_space=pl.ANY),
            pl.BlockSpec(memory_space=pl.ANY),
            pl.BlockSpec((1, out_pad), lambda c: (0, 0)),
            pl.BlockSpec((1, out_pad), lambda c: (0, 0)),
            pl.BlockSpec((1, out_pad), lambda c: (0, 0)),
        ],
        out_specs=pl.BlockSpec(memory_space=pl.ANY),
        scratch_shapes=[
            pltpu.VMEM((2, tm, in_pad), jnp.float32),
            pltpu.VMEM((2, tm, out_pad), jnp.float32),
            pltpu.VMEM((in_pad, out_pad), jnp.float32),
            pltpu.SemaphoreType.DMA((2,)),
            pltpu.SemaphoreType.DMA((2,)),
            pltpu.SemaphoreType.DMA,
        ],
        compiler_params=pltpu.CompilerParams(
            dimension_semantics=("parallel",),
            vmem_limit_bytes=64 * 1024 * 1024,
        ),
    )(xp, wp, bp, gp, betap)

    if (n_pad, out_pad) != (n, out_dim):
        y = y[:n, :out_dim]
    return y


# speedup vs baseline: 1.1393x; 1.1393x over previous
import functools

import jax
import jax.numpy as jnp
from jax.experimental import pallas as pl
from jax.experimental.pallas import tpu as pltpu


def _round_up(v, m):
    return ((v + m - 1) // m) * m


def _fused_kernel(x_ref, w_ref, b_ref, g_ref, beta_ref, o_ref, *, eps,
                  true_out_dim):
    y = jnp.dot(x_ref[...], w_ref[...], preferred_element_type=jnp.float32)

    z = (y + b_ref[...]).astype(jnp.bfloat16)

    inv_d = 1.0 / float(true_out_dim)
    s1 = jnp.sum(z, axis=-1, keepdims=True, dtype=jnp.float32)
    zf = z.astype(jnp.float32)
    s2 = jnp.sum(zf * zf, axis=-1, keepdims=True)
    mean = s1 * inv_d
    var = jnp.maximum(s2 * inv_d - mean * mean, 0.0)
    inv = jax.lax.rsqrt(var + eps)

    out = (zf - mean) * inv * g_ref[...] + beta_ref[...]
    o_ref[...] = jnp.maximum(out, 0.0).astype(o_ref.dtype)


def kernel(x, w, b, gamma, beta, *, eps=1e-5):
    n, in_dim = x.shape
    out_dim = w.shape[1]

    in_pad = _round_up(in_dim, 128)
    out_pad = _round_up(out_dim, 128)
    tm = min(1024, _round_up(n, 8))
    n_pad = _round_up(n, tm)

    xp = x
    if (n_pad, in_pad) != x.shape:
        xp = jnp.zeros((n_pad, in_pad), x.dtype).at[:n, :in_dim].set(x)
    bp = b.astype(jnp.float32)
    gp = gamma.astype(jnp.float32)
    betap = beta.astype(jnp.float32)
    if out_pad != out_dim:
        bp = jnp.zeros((1, out_pad), jnp.float32).at[:, :out_dim].set(bp)
        gp = jnp.ones((1, out_pad), jnp.float32).at[:, :out_dim].set(gp)
        betap = jnp.zeros((1, out_pad), jnp.float32).at[:, :out_dim].set(betap)

    wp = w
    if (in_pad, out_pad) != w.shape:
        wp = jnp.zeros((in_pad, out_pad), w.dtype).at[:in_dim, :out_dim].set(w)

    body = functools.partial(_fused_kernel, eps=eps, true_out_dim=out_dim)
    y = pl.pallas_call(
        body,
        out_shape=jax.ShapeDtypeStruct((n_pad, out_pad), x.dtype),
        grid=(n_pad // tm,),
        in_specs=[
            pl.BlockSpec((tm, in_pad), lambda m: (m, 0)),
            pl.BlockSpec((in_pad, out_pad), lambda m: (0, 0)),
            pl.BlockSpec((1, out_pad), lambda m: (0, 0)),
            pl.BlockSpec((1, out_pad), lambda m: (0, 0)),
            pl.BlockSpec((1, out_pad), lambda m: (0, 0)),
        ],
        out_specs=pl.BlockSpec((tm, out_pad), lambda m: (m, 0)),
        compiler_params=pltpu.CompilerParams(
            dimension_semantics=("parallel",),
            vmem_limit_bytes=64 * 1024 * 1024,
        ),
    )(xp, wp, bp, gp, betap)

    if (n_pad, out_pad) != (n, out_dim):
        y = y[:n, :out_dim]
    return y


# speedup vs baseline: 1.1531x; 1.0120x over previous
import functools

import jax
import jax.numpy as jnp
from jax.experimental import pallas as pl
from jax.experimental.pallas import tpu as pltpu


def _round_up(v, m):
    return ((v + m - 1) // m) * m


def _fused_kernel(x_ref, w_ref, b_ref, g_ref, beta_ref, o_ref, *, eps,
                  true_out_dim):
    xb = x_ref[...].astype(jnp.bfloat16)
    wb = w_ref[...].astype(jnp.bfloat16)
    y = jnp.dot(xb, wb, preferred_element_type=jnp.float32)

    y = y + b_ref[...]

    inv_d = 1.0 / float(true_out_dim)
    s1 = jnp.sum(y, axis=-1, keepdims=True)
    s2 = jnp.sum(y * y, axis=-1, keepdims=True)
    mean = s1 * inv_d
    var = jnp.maximum(s2 * inv_d - mean * mean, 0.0)
    inv = jax.lax.rsqrt(var + eps)

    out = (y - mean) * inv * g_ref[...] + beta_ref[...]
    o_ref[...] = jnp.maximum(out, 0.0).astype(o_ref.dtype)


def kernel(x, w, b, gamma, beta, *, eps=1e-5):
    n, in_dim = x.shape
    out_dim = w.shape[1]

    in_pad = _round_up(in_dim, 128)
    out_pad = _round_up(out_dim, 128)
    tm = min(1024, _round_up(n, 8))
    n_pad = _round_up(n, tm)

    xp = x
    if (n_pad, in_pad) != x.shape:
        xp = jnp.zeros((n_pad, in_pad), x.dtype).at[:n, :in_dim].set(x)
    bp = b.astype(jnp.float32)
    gp = gamma.astype(jnp.float32)
    betap = beta.astype(jnp.float32)
    if out_pad != out_dim:
        bp = jnp.zeros((1, out_pad), jnp.float32).at[:, :out_dim].set(bp)
        gp = jnp.ones((1, out_pad), jnp.float32).at[:, :out_dim].set(gp)
        betap = jnp.zeros((1, out_pad), jnp.float32).at[:, :out_dim].set(betap)

    wp = w
    if (in_pad, out_pad) != w.shape:
        wp = jnp.zeros((in_pad, out_pad), w.dtype).at[:in_dim, :out_dim].set(w)

    body = functools.partial(_fused_kernel, eps=eps, true_out_dim=out_dim)
    y = pl.pallas_call(
        body,
        out_shape=jax.ShapeDtypeStruct((n_pad, out_pad), x.dtype),
        grid=(n_pad // tm,),
        in_specs=[
            pl.BlockSpec((tm, in_pad), lambda m: (m, 0)),
            pl.BlockSpec((in_pad, out_pad), lambda m: (0, 0)),
            pl.BlockSpec((1, out_pad), lambda m: (0, 0)),
            pl.BlockSpec((1, out_pad), lambda m: (0, 0)),
            pl.BlockSpec((1, out_pad), lambda m: (0, 0)),
        ],
        out_specs=pl.BlockSpec((tm, out_pad), lambda m: (m, 0)),
        compiler_params=pltpu.CompilerParams(
            dimension_semantics=("parallel",),
            vmem_limit_bytes=64 * 1024 * 1024,
        ),
    )(xp, wp, bp, gp, betap)

    if (n_pad, out_pad) != (n, out_dim):
        y = y[:n, :out_dim]
    return y


# speedup vs baseline: 1.1539x; 1.0008x over previous
import functools

import jax
import jax.numpy as jnp
from jax.experimental import pallas as pl
from jax.experimental.pallas import tpu as pltpu


def _round_up(v, m):
    return ((v + m - 1) // m) * m


def _fused_kernel(x_ref, w_ref, b_ref, g_ref, beta_ref, o_ref, *, eps,
                  true_out_dim):
    xb = x_ref[...].astype(jnp.bfloat16)
    wb = w_ref[...].astype(jnp.bfloat16)
    y = jnp.dot(xb, wb, preferred_element_type=jnp.float32)

    y = y + b_ref[...]

    inv_d = 1.0 / float(true_out_dim)
    s1 = jnp.sum(y, axis=-1, keepdims=True)
    s2 = jnp.sum(y * y, axis=-1, keepdims=True)
    mean = s1 * inv_d
    var = jnp.maximum(s2 * inv_d - mean * mean, 0.0)
    inv = jax.lax.rsqrt(var + eps)

    out = (y - mean) * inv * g_ref[...] + beta_ref[...]
    o_ref[...] = jnp.maximum(out, 0.0).astype(o_ref.dtype)


def kernel(x, w, b, gamma, beta, *, eps=1e-5):
    n, in_dim = x.shape
    out_dim = w.shape[1]

    in_pad = _round_up(in_dim, 128)
    out_pad = _round_up(out_dim, 128)
    tm = min(1024, _round_up(n, 8))
    n_pad = _round_up(n, tm)

    xp = x
    if (n_pad, in_pad) != x.shape:
        xp = jnp.zeros((n_pad, in_pad), x.dtype).at[:n, :in_dim].set(x)
    bp = b.astype(jnp.float32)
    gp = gamma.astype(jnp.float32)
    betap = beta.astype(jnp.float32)
    if out_pad != out_dim:
        bp = jnp.zeros((1, out_pad), jnp.float32).at[:, :out_dim].set(bp)
        gp = jnp.ones((1, out_pad), jnp.float32).at[:, :out_dim].set(gp)
        betap = jnp.zeros((1, out_pad), jnp.float32).at[:, :out_dim].set(betap)

    wp = w
    if (in_pad, out_pad) != w.shape:
        wp = jnp.zeros((in_pad, out_pad), w.dtype).at[:in_dim, :out_dim].set(w)

    body = functools.partial(_fused_kernel, eps=eps, true_out_dim=out_dim)
    y = pl.pallas_call(
        body,
        out_shape=jax.ShapeDtypeStruct((n_pad, out_pad), x.dtype),
        grid=(n_pad // tm,),
        in_specs=[
            pl.BlockSpec((tm, in_pad), lambda m: (m, 0)),
            pl.BlockSpec((in_pad, out_pad), lambda m: (0, 0)),
            pl.BlockSpec((1, out_pad), lambda m: (0, 0)),
            pl.BlockSpec((1, out_pad), lambda m: (0, 0)),
            pl.BlockSpec((1, out_pad), lambda m: (0, 0)),
        ],
        out_specs=pl.BlockSpec((tm, out_pad), lambda m: (m, 0)),
        compiler_params=pltpu.CompilerParams(
            dimension_semantics=("parallel",),
            vmem_limit_bytes=100 * 1024 * 1024,
        ),
    )(xp, wp, bp, gp, betap)

    if (n_pad, out_pad) != (n, out_dim):
        y = y[:n, :out_dim]
    return y
